# baseline (speedup 1.0000x reference)
import jax
import jax.numpy as jnp
from jax import lax
from jax.experimental import pallas as pl
from jax.experimental.pallas import tpu as pltpu
from jax.experimental.pallas import tpu_sc as plsc

N_NODES = 10000
D = 128
NW = 16
C = 128
K = 80
K0 = 112
K1 = 48
NC = 2
NS = 16
NBUF = 2
E_PAD = NC * NS * K * C


def _tab_body(w_ref, x_ref, o_ref):
    o_ref[0] = jnp.tanh(w_ref[pl.program_id(1), 0] * x_ref[...])


_tc_tab = pl.pallas_call(
    _tab_body,
    out_shape=jax.ShapeDtypeStruct((NW, N_NODES, D), jnp.float32),
    grid=(10, NW),
    in_specs=[pl.BlockSpec(memory_space=pltpu.SMEM),
              pl.BlockSpec((1000, D), lambda j, i: (j, 0))],
    out_specs=pl.BlockSpec((1, 1000, D), lambda j, i: (i, j, 0)),
)


def _sc_body(t_hbm, fidx_hbm, dst_hbm, out_hbm,
             srcv, dstv, rows, acc):
    c = lax.axis_index("c")
    s = lax.axis_index("s")
    myk = jnp.where(c == 0, K0, K1)

    @pl.when(c == 0)
    def _stage0():
        pltpu.sync_copy(fidx_hbm.at[pl.ds(s * K0, K0)], srcv)
        pltpu.sync_copy(dst_hbm.at[pl.ds(s * K0, K0)], dstv)

    @pl.when(c == 1)
    def _stage1():
        o = NS * K0 + s * K1
        pltpu.sync_copy(fidx_hbm.at[pl.ds(o, K1)], srcv.at[pl.ds(0, K1)])
        pltpu.sync_copy(dst_hbm.at[pl.ds(o, K1)], dstv.at[pl.ds(0, K1)])

    def _zero_row(e, _):
        for k in range(8):
            rows[0, e, pl.ds(k * 16, 16)] = jnp.zeros((16,), jnp.float32)
        return 0
    lax.fori_loop(0, C, _zero_row, 0)
    base = s * 624
    for i in range(4):
        pltpu.sync_copy(rows.at[0], acc.at[pl.ds(base + i * C, C)])
    pltpu.sync_copy(rows.at[0, pl.ds(0, 112)], acc.at[pl.ds(base + 512, 112)])

    @pl.when(s == 0)
    def _zero_tail():
        pltpu.sync_copy(rows.at[0, pl.ds(0, 16)], acc.at[pl.ds(9984, 16)])

    def _fold(j, _):
        for g in range(C // 16):
            sl = pl.ds(g * 16, 16)
            v = srcv[j, sl]
            srcv[j, sl] = (v >> 14) * N_NODES + (v & 16383)
        return 0
    lax.fori_loop(0, myk, _fold, 0)
    plsc.subcore_barrier()

    def _chunk(j, _):
        pltpu.sync_copy(t_hbm.at[srcv.at[j]], rows.at[0])
        pltpu.sync_copy(rows.at[0], acc.at[dstv.at[j]], add=True)
        return 0
    lax.fori_loop(0, myk, _chunk, 0)

    plsc.subcore_barrier()
    pltpu.sync_copy(acc.at[pl.ds(base, 624)],
                    out_hbm.at[c, pl.ds(base, 624)])

    @pl.when(s == 0)
    def _write_tail():
        pltpu.sync_copy(acc.at[pl.ds(9984, 16)],
                        out_hbm.at[c, pl.ds(9984, 16)])


_sc_call = pl.kernel(
    _sc_body,
    out_type=jax.ShapeDtypeStruct((NC, N_NODES, D), jnp.float32),
    mesh=plsc.VectorSubcoreMesh(core_axis_name="c", subcore_axis_name="s"),
    scratch_types=[
        pltpu.VMEM((K0, C), jnp.int32),
        pltpu.VMEM((K0, C), jnp.int32),
        pltpu.VMEM((1, C, D), jnp.float32),
        pltpu.VMEM_SHARED((N_NODES + 8, D), jnp.float32),
    ],
)


def _add_body(p_ref, o_ref):
    o_ref[...] = p_ref[0] + p_ref[1]


_tc_add = pl.pallas_call(
    _add_body,
    out_shape=jax.ShapeDtypeStruct((N_NODES, D), jnp.float32),
    grid=(10,),
    in_specs=[pl.BlockSpec((2, 1000, D), lambda i: (0, i, 0))],
    out_specs=pl.BlockSpec((1000, D), lambda i: (i, 0)),
)


@jax.jit
def kernel(x, edge_index, edge_weight_idx, weights):
    e = edge_index.shape[1]
    pad = E_PAD - e
    src = jnp.concatenate([edge_index[0], jnp.zeros((pad,), jnp.int32)])
    dst = jnp.concatenate([edge_index[1],
                           jnp.full((pad,), N_NODES, jnp.int32)])
    wix = jnp.concatenate([edge_weight_idx, jnp.zeros((pad,), jnp.int32)])
    comb2 = (wix * 16384 + src).reshape(NC * NS * K, C)
    dst2 = dst.reshape(NC * NS * K, C)
    tab = _tc_tab(weights.reshape(NW, 1), x).reshape(NW * N_NODES, D)
    partials = _sc_call(tab, comb2, dst2)
    return _tc_add(partials)


# speedup vs baseline: 1.0223x; 1.0223x over previous
import jax
import jax.numpy as jnp
from jax import lax
from jax.experimental import pallas as pl
from jax.experimental.pallas import tpu as pltpu
from jax.experimental.pallas import tpu_sc as plsc

N_NODES = 10000
D = 128
NW = 16
C = 128
K = 80
NC = 2
NS = 16
NBUF = 2
E_PAD = NC * NS * K * C


def _tab_body(w_ref, x_ref, o_ref):
    o_ref[0] = jnp.tanh(w_ref[pl.program_id(1), 0] * x_ref[...])


_tc_tab = pl.pallas_call(
    _tab_body,
    out_shape=jax.ShapeDtypeStruct((NW, N_NODES, D), jnp.float32),
    grid=(10, NW),
    in_specs=[pl.BlockSpec(memory_space=pltpu.SMEM),
              pl.BlockSpec((1000, D), lambda j, i: (j, 0))],
    out_specs=pl.BlockSpec((1, 1000, D), lambda j, i: (i, j, 0)),
)


def _sc_body(t_hbm, fidx_hbm, dst_hbm, wix_hbm, out_hbm,
             srcv, dstv, wiv, rows, acc):
    c = lax.axis_index("c")
    s = lax.axis_index("s")
    b = c * NS + s

    pltpu.sync_copy(fidx_hbm.at[b], srcv)
    pltpu.sync_copy(dst_hbm.at[b], dstv)
    pltpu.sync_copy(wix_hbm.at[b], wiv)

    def _zero_row(e, _):
        for k in range(8):
            rows[0, e, pl.ds(k * 16, 16)] = jnp.zeros((16,), jnp.float32)
        return 0
    lax.fori_loop(0, C, _zero_row, 0)
    base = s * 624
    for i in range(4):
        pltpu.sync_copy(rows.at[0], acc.at[pl.ds(base + i * C, C)])
    pltpu.sync_copy(rows.at[0, pl.ds(0, 112)], acc.at[pl.ds(base + 512, 112)])

    @pl.when(s == 0)
    def _zero_tail():
        pltpu.sync_copy(rows.at[0, pl.ds(0, 16)], acc.at[pl.ds(9984, 16)])

    def _fold(j, _):
        for g in range(C // 16):
            sl = pl.ds(g * 16, 16)
            srcv[j, sl] = srcv[j, sl] + wiv[j, sl] * N_NODES
        return 0
    lax.fori_loop(0, K, _fold, 0)
    plsc.subcore_barrier()

    def _chunk(j, _):
        pltpu.sync_copy(t_hbm.at[srcv.at[j]], rows.at[0])
        pltpu.sync_copy(rows.at[0], acc.at[dstv.at[j]], add=True)
        return 0
    lax.fori_loop(0, K, _chunk, 0)

    plsc.subcore_barrier()
    pltpu.sync_copy(acc.at[pl.ds(base, 624)],
                    out_hbm.at[c, pl.ds(base, 624)])

    @pl.when(s == 0)
    def _write_tail():
        pltpu.sync_copy(acc.at[pl.ds(9984, 16)],
                        out_hbm.at[c, pl.ds(9984, 16)])


_sc_call = pl.kernel(
    _sc_body,
    out_type=jax.ShapeDtypeStruct((NC, N_NODES, D), jnp.float32),
    mesh=plsc.VectorSubcoreMesh(core_axis_name="c", subcore_axis_name="s"),
    scratch_types=[
        pltpu.VMEM((K, C), jnp.int32),
        pltpu.VMEM((K, C), jnp.int32),
        pltpu.VMEM((K, C), jnp.int32),
        pltpu.VMEM((1, C, D), jnp.float32),
        pltpu.VMEM_SHARED((N_NODES + 8, D), jnp.float32),
    ],
)


def _add_body(p_ref, o_ref):
    o_ref[...] = p_ref[0] + p_ref[1]


_tc_add = pl.pallas_call(
    _add_body,
    out_shape=jax.ShapeDtypeStruct((N_NODES, D), jnp.float32),
    grid=(10,),
    in_specs=[pl.BlockSpec((2, 1000, D), lambda i: (0, i, 0))],
    out_specs=pl.BlockSpec((1000, D), lambda i: (i, 0)),
)


@jax.jit
def kernel(x, edge_index, edge_weight_idx, weights):
    e = edge_index.shape[1]
    pad = E_PAD - e
    src = jnp.concatenate([edge_index[0], jnp.zeros((pad,), jnp.int32)])
    dst = jnp.concatenate([edge_index[1],
                           jnp.full((pad,), N_NODES, jnp.int32)])
    wix = jnp.concatenate([edge_weight_idx, jnp.zeros((pad,), jnp.int32)])
    src3 = src.reshape(NC * NS, K, C)
    dst3 = dst.reshape(NC * NS, K, C)
    wix3 = wix.reshape(NC * NS, K, C)
    tab = _tc_tab(weights.reshape(NW, 1), x).reshape(NW * N_NODES, D)
    partials = _sc_call(tab, src3, dst3, wix3)
    return _tc_add(partials)
